# Initial kernel scaffold; baseline (speedup 1.0000x reference)
import jax
import jax.numpy as jnp
from jax.experimental import pallas as pl


def kernel(x, W):
    raise NotImplementedError("write your pallas kernel here")


# speedup vs baseline: 1.7555x; 1.7555x over previous
import jax
import jax.numpy as jnp
from jax.experimental import pallas as pl
from jax.experimental.pallas import tpu as pltpu

D_MODEL = 2048
NUM_EXPERTS = 64
TOP_K = 2
Z_LOSS_COEF = 0.001
BALANCE_LOSS_COEF = 0.01

BLK = 1024


def _router_kernel(x_ref, w_ref, idx_ref, wgt_ref, loss_ref,
                   usage_ref, zacc_ref):
    step = pl.program_id(0)
    nsteps = pl.num_programs(0)

    @pl.when(step == 0)
    def _init():
        usage_ref[...] = jnp.zeros_like(usage_ref)
        zacc_ref[0] = jnp.float32(0.0)

    logits = jnp.dot(x_ref[...], w_ref[...],
                     preferred_element_type=jnp.float32)

    lane = jax.lax.broadcasted_iota(jnp.int32, logits.shape, 1)
    big = jnp.int32(NUM_EXPERTS)

    m1 = jnp.max(logits, axis=-1, keepdims=True)
    i1 = jnp.min(jnp.where(logits == m1, lane, big), axis=-1, keepdims=True)
    masked = jnp.where(lane == i1, -jnp.inf, logits)
    m2 = jnp.max(masked, axis=-1, keepdims=True)
    i2 = jnp.min(jnp.where(masked == m2, lane, big), axis=-1, keepdims=True)

    d = jnp.exp(m2 - m1)
    w1 = 1.0 / (1.0 + d)
    w2 = d / (1.0 + d)

    idx_ref[...] = jnp.concatenate([i1, i2], axis=1)
    wgt_ref[...] = jnp.concatenate([w1, w2], axis=1)

    u1 = jnp.sum(jnp.where(lane == i1, w1, 0.0), axis=0, keepdims=True)
    u2 = jnp.sum(jnp.where(lane == i2, w2, 0.0), axis=0, keepdims=True)
    usage_ref[...] += jnp.concatenate([u1, u2], axis=0)

    lse = m1[:, 0] + jnp.log(jnp.sum(jnp.exp(logits - m1), axis=-1))
    zacc_ref[0] += jnp.sum(lse * lse)

    @pl.when(step == nsteps - 1)
    def _fin():
        u = usage_ref[...]
        mean = jnp.mean(u)
        std = jnp.sqrt(jnp.mean((u - mean) * (u - mean)))
        bal = std / mean * BALANCE_LOSS_COEF
        n_rows = nsteps * BLK
        z = zacc_ref[0] / n_rows * Z_LOSS_COEF
        loss_ref[...] = jnp.reshape(bal + z, (1, 1))


def kernel(x, W):
    B, S, D = x.shape
    rows = B * S
    x2 = x.reshape(rows, D)
    grid = (rows // BLK,)

    idx, wgt, loss = pl.pallas_call(
        _router_kernel,
        grid=grid,
        in_specs=[
            pl.BlockSpec((BLK, D), lambda i: (i, 0)),
            pl.BlockSpec((D, NUM_EXPERTS), lambda i: (0, 0)),
        ],
        out_specs=[
            pl.BlockSpec((BLK, TOP_K), lambda i: (i, 0)),
            pl.BlockSpec((BLK, TOP_K), lambda i: (i, 0)),
            pl.BlockSpec((1, 1), lambda i: (0, 0)),
        ],
        out_shape=[
            jax.ShapeDtypeStruct((rows, TOP_K), jnp.int32),
            jax.ShapeDtypeStruct((rows, TOP_K), jnp.float32),
            jax.ShapeDtypeStruct((1, 1), jnp.float32),
        ],
        scratch_shapes=[
            pltpu.VMEM((TOP_K, NUM_EXPERTS), jnp.float32),
            pltpu.SMEM((1,), jnp.float32),
        ],
        compiler_params=pltpu.CompilerParams(
            dimension_semantics=("arbitrary",),
        ),
    )(x2, W)

    return (idx.reshape(B, S, TOP_K), wgt.reshape(B, S, TOP_K),
            loss[0, 0])


# speedup vs baseline: 1.8305x; 1.0427x over previous
import jax
import jax.numpy as jnp
from jax.experimental import pallas as pl
from jax.experimental.pallas import tpu as pltpu

D_MODEL = 2048
NUM_EXPERTS = 64
TOP_K = 2
Z_LOSS_COEF = 0.001
BALANCE_LOSS_COEF = 0.01

BLK = 2048


def _router_kernel(x_ref, w_ref, idx_ref, wgt_ref, loss_ref,
                   usage_ref, zacc_ref):
    step = pl.program_id(0)
    nsteps = pl.num_programs(0)

    @pl.when(step == 0)
    def _init():
        usage_ref[...] = jnp.zeros_like(usage_ref)
        zacc_ref[0] = jnp.float32(0.0)

    logits = jnp.dot(x_ref[...], w_ref[...],
                     preferred_element_type=jnp.float32)

    lane = jax.lax.broadcasted_iota(jnp.int32, logits.shape, 1)
    big = jnp.int32(NUM_EXPERTS)

    m1 = jnp.max(logits, axis=-1, keepdims=True)
    i1 = jnp.min(jnp.where(logits == m1, lane, big), axis=-1, keepdims=True)
    masked = jnp.where(lane == i1, -jnp.inf, logits)
    m2 = jnp.max(masked, axis=-1, keepdims=True)
    i2 = jnp.min(jnp.where(masked == m2, lane, big), axis=-1, keepdims=True)

    d = jnp.exp(m2 - m1)
    w1 = 1.0 / (1.0 + d)
    w2 = d / (1.0 + d)

    idx_ref[...] = jnp.concatenate([i1, i2], axis=1)
    wgt_ref[...] = jnp.concatenate([w1, w2], axis=1)

    u1 = jnp.sum(jnp.where(lane == i1, w1, 0.0), axis=0, keepdims=True)
    u2 = jnp.sum(jnp.where(lane == i2, w2, 0.0), axis=0, keepdims=True)
    usage_ref[...] += jnp.concatenate([u1, u2], axis=0)

    lse = m1[:, 0] + jnp.log(jnp.sum(jnp.exp(logits - m1), axis=-1))
    zacc_ref[0] += jnp.sum(lse * lse)

    @pl.when(step == nsteps - 1)
    def _fin():
        u = usage_ref[...]
        mean = jnp.mean(u)
        std = jnp.sqrt(jnp.mean((u - mean) * (u - mean)))
        bal = std / mean * BALANCE_LOSS_COEF
        n_rows = nsteps * BLK
        z = zacc_ref[0] / n_rows * Z_LOSS_COEF
        loss_ref[...] = jnp.reshape(bal + z, (1, 1))


def kernel(x, W):
    B, S, D = x.shape
    rows = B * S
    x2 = x.reshape(rows, D)
    grid = (rows // BLK,)

    idx, wgt, loss = pl.pallas_call(
        _router_kernel,
        grid=grid,
        in_specs=[
            pl.BlockSpec((BLK, D), lambda i: (i, 0)),
            pl.BlockSpec((D, NUM_EXPERTS), lambda i: (0, 0)),
        ],
        out_specs=[
            pl.BlockSpec((BLK, TOP_K), lambda i: (i, 0)),
            pl.BlockSpec((BLK, TOP_K), lambda i: (i, 0)),
            pl.BlockSpec((1, 1), lambda i: (0, 0)),
        ],
        out_shape=[
            jax.ShapeDtypeStruct((rows, TOP_K), jnp.int32),
            jax.ShapeDtypeStruct((rows, TOP_K), jnp.float32),
            jax.ShapeDtypeStruct((1, 1), jnp.float32),
        ],
        scratch_shapes=[
            pltpu.VMEM((TOP_K, NUM_EXPERTS), jnp.float32),
            pltpu.SMEM((1,), jnp.float32),
        ],
        compiler_params=pltpu.CompilerParams(
            dimension_semantics=("arbitrary",),
        ),
    )(x2, W)

    return (idx.reshape(B, S, TOP_K), wgt.reshape(B, S, TOP_K),
            loss[0, 0])
